# baseline (speedup 1.0000x reference)
import functools

import jax
import jax.numpy as jnp
from jax import lax
from jax.experimental import pallas as pl
from jax.experimental.pallas import tpu as pltpu
from jax.experimental.pallas import tpu_sc as plsc

NC = 2
NS = 16
NW = NC * NS
F = 16

_MESH = plsc.VectorSubcoreMesh(
    core_axis_name="c", subcore_axis_name="s", num_cores=NC, num_subcores=NS
)
_SC_PARAMS = pltpu.CompilerParams(use_tc_tiling_on_sc=False)


def _zero_rows(buf, nrows):
    zero = jnp.zeros((F,), jnp.float32)

    def body(j, _):
        buf[j] = zero
        return 0

    lax.fori_loop(0, nrows, body, 0, unroll=4)


def _make_agg_kernel(n_nodes, n_edges, chunk):
    epw = n_edges // NW
    nchunk = epw // chunk
    rpt = -(-n_nodes // (NS * 8)) * 8
    n_pad = rpt * NS

    def body(table, ei, out, sidx0, didx0, rows0, sidx1, didx1, rows1,
             tbuf, acc, gsem0, gsem1, ssem0, ssem1):
        c = lax.axis_index("c")
        s = lax.axis_index("s")
        wid = s * NC + c
        _zero_rows(tbuf, rpt)
        pltpu.sync_copy(tbuf, acc.at[pl.ds(s * rpt, rpt)])
        plsc.subcore_barrier()
        base = wid * epw
        bufs = [(sidx0, didx0, rows0, gsem0, ssem0),
                (sidx1, didx1, rows1, gsem1, ssem1)]

        def start_gather(k):
            sidx, didx, rows, gsem, _ = bufs[k % 2]
            off = base + k * chunk
            pltpu.sync_copy(ei.at[0].at[pl.ds(off, chunk)], sidx)
            pltpu.sync_copy(ei.at[1].at[pl.ds(off, chunk)], didx)
            return pltpu.async_copy(table.at[sidx], rows, gsem)

        gathers = {0: start_gather(0)}
        scatters = {}
        for k in range(nchunk):
            sidx, didx, rows, gsem, ssem = bufs[k % 2]
            if k + 1 < nchunk:
                if k - 1 >= 0:
                    scatters.pop(k - 1).wait()
                gathers[k + 1] = start_gather(k + 1)
            gathers.pop(k).wait()
            scatters[k] = pltpu.async_copy(rows, acc.at[didx], ssem, add=True)
        for d in scatters.values():
            d.wait()
        plsc.subcore_barrier()
        pltpu.sync_copy(acc.at[pl.ds(s * rpt, rpt)], tbuf)
        pltpu.sync_copy(tbuf, out.at[c].at[pl.ds(s * rpt, rpt)])

    return pl.kernel(
        body,
        out_type=jax.ShapeDtypeStruct((NC, n_pad, F), jnp.float32),
        mesh=_MESH,
        compiler_params=_SC_PARAMS,
        scratch_types=[
            pltpu.VMEM((chunk,), jnp.int32),
            pltpu.VMEM((chunk,), jnp.int32),
            pltpu.VMEM((chunk, F), jnp.float32),
            pltpu.VMEM((chunk,), jnp.int32),
            pltpu.VMEM((chunk,), jnp.int32),
            pltpu.VMEM((chunk, F), jnp.float32),
            pltpu.VMEM((rpt, F), jnp.float32),
            pltpu.VMEM_SHARED((n_pad, F), jnp.float32),
            pltpu.SemaphoreType.DMA,
            pltpu.SemaphoreType.DMA,
            pltpu.SemaphoreType.DMA,
            pltpu.SemaphoreType.DMA,
        ],
    )


DEGW = 8


def _make_deg_kernel(n_nodes, n_edges, chunk):
    epw = n_edges // NW
    nchunk = epw // chunk
    rpt = -(-n_nodes // (NS * 8)) * 8
    n_pad = rpt * NS

    def body(ei, ones, zeros, out, didx0, didx1, rows, tbuf, acc,
             ssem0, ssem1):
        c = lax.axis_index("c")
        s = lax.axis_index("s")
        wid = s * NC + c
        pltpu.sync_copy(zeros, tbuf)
        pltpu.sync_copy(tbuf, acc.at[pl.ds(s * rpt, rpt)])
        pltpu.sync_copy(ones, rows)
        plsc.subcore_barrier()
        base = wid * epw
        bufs = [(didx0, ssem0), (didx1, ssem1)]

        def load_idx(k):
            didx, _ = bufs[k % 2]
            pltpu.sync_copy(ei.at[1].at[pl.ds(base + k * chunk, chunk)], didx)

        load_idx(0)
        scatters = {}
        for k in range(nchunk):
            didx, ssem = bufs[k % 2]
            if k + 1 < nchunk:
                if k - 1 >= 0:
                    scatters.pop(k - 1).wait()
                load_idx(k + 1)
            scatters[k] = pltpu.async_copy(rows, acc.at[didx], ssem, add=True)
        for d in scatters.values():
            d.wait()
        plsc.subcore_barrier()
        pltpu.sync_copy(acc.at[pl.ds(s * rpt, rpt)], tbuf)
        pltpu.sync_copy(tbuf, out.at[c].at[pl.ds(s * rpt, rpt)])

    return pl.kernel(
        body,
        out_type=jax.ShapeDtypeStruct((NC, n_pad, DEGW), jnp.float32),
        mesh=_MESH,
        compiler_params=_SC_PARAMS,
        scratch_types=[
            pltpu.VMEM((chunk,), jnp.int32),
            pltpu.VMEM((chunk,), jnp.int32),
            pltpu.VMEM((chunk, DEGW), jnp.float32),
            pltpu.VMEM((rpt, DEGW), jnp.float32),
            pltpu.VMEM_SHARED((n_pad, DEGW), jnp.float32),
            pltpu.SemaphoreType.DMA,
            pltpu.SemaphoreType.DMA,
        ],
    )


def _tc1a_body(x8_ref, w1bd_ref, xw_ref):
    xw_ref[...] = jnp.dot(x8_ref[...], w1bd_ref[...],
                          preferred_element_type=jnp.float32)


def _tc1b_body(nvh, degp8_ref, xw_ref, plo_ref, phi_ref, t1_ref, dinv_ref):
    deg8 = degp8_ref[0, :nvh] + degp8_ref[1, :nvh] + 1.0
    d8 = lax.rsqrt(deg8)
    lo = jnp.dot(d8, plo_ref[...], preferred_element_type=jnp.float32)
    hi = jnp.dot(d8, phi_ref[...], preferred_element_type=jnp.float32)
    dinv_v = jnp.concatenate([lo[:, None, :], hi[:, None, :]], axis=1)
    dinv_v = dinv_v.reshape(2 * nvh, 128)
    dinv_ref[...] = dinv_v
    t1_ref[...] = xw_ref[...] * dinv_v


def _tc2_body(nv, aggp_ref, t1_ref, dinv_ref, b1_ref, t2_ref):
    dinv_v = dinv_ref[...]
    s_v = aggp_ref[0, :nv] + aggp_ref[1, :nv] + t1_ref[...]
    h_v = jnp.maximum(dinv_v * s_v + b1_ref[...], 0.0)
    t2_ref[...] = h_v * dinv_v


def _tc3_body(nv, aggp_ref, t2_ref, dinv_ref, w2bd_ref, b2_ref, sel_ref,
              selt_ref, out_ref):
    agg_v = dinv_ref[...] * (aggp_ref[0, :nv] + aggp_ref[1, :nv] + t2_ref[...])
    logits = (
        jnp.dot(agg_v, w2bd_ref[...], preferred_element_type=jnp.float32)
        + b2_ref[...]
    )
    m = jnp.max(logits, axis=1, keepdims=True)
    z = logits - m
    gsum = jnp.dot(jnp.exp(z), sel_ref[...], preferred_element_type=jnp.float32)
    lse = jnp.dot(jnp.log(gsum), selt_ref[...],
                  preferred_element_type=jnp.float32)
    z2 = z - lse
    gsum2 = jnp.dot(jnp.exp(z2), sel_ref[...],
                    preferred_element_type=jnp.float32)
    lse2 = jnp.dot(jnp.log(gsum2), selt_ref[...],
                   preferred_element_type=jnp.float32)
    out_ref[...] = z2 - lse2


def _block_diag(w, copies):
    d_in, d_out = w.shape
    out = jnp.zeros((copies * d_in, copies * d_out), w.dtype)
    for j in range(copies):
        out = out.at[j * d_in:(j + 1) * d_in, j * d_out:(j + 1) * d_out].set(w)
    return out


@jax.jit
def kernel(x, edge_index, W1, b1, W2, b2):
    n, d_in = x.shape
    d_hid = W1.shape[1]
    d_out = W2.shape[1]
    e = edge_index.shape[1]
    nv = n // 8

    ei = edge_index.astype(jnp.int32)

    chunk = 2000
    deg_call = _make_deg_kernel(n, e, chunk)
    agg_call = _make_agg_kernel(n, e, chunk)
    rpt = -(-n // (NS * 8)) * 8
    n_pad = rpt * NS
    npv = n_pad // 8
    nvh = n // 16

    ones8 = jnp.ones((chunk, DEGW), jnp.float32)
    zeros8 = jnp.zeros((rpt, DEGW), jnp.float32)
    degp8_v = deg_call(ei, ones8, zeros8).reshape(NC, n_pad * DEGW // 128, 128)

    w1p = jnp.zeros((d_in, F), W1.dtype).at[:, :d_hid].set(W1)
    w1bd = _block_diag(w1p, 8)
    x8 = x.reshape(nv, 8 * d_in)
    xw_v = pl.pallas_call(
        _tc1a_body,
        out_shape=jax.ShapeDtypeStruct((nv, 8 * F), jnp.float32),
    )(x8, w1bd)

    c128 = jnp.arange(128)
    plo = jnp.zeros((128, 128), jnp.float32).at[(c128 // 16) * 8, c128].set(1.0)
    phi = jnp.zeros((128, 128), jnp.float32).at[
        (c128 // 16) * 8 + 64, c128].set(1.0)
    t1_v, dinv_v = pl.pallas_call(
        functools.partial(_tc1b_body, nvh),
        out_shape=[
            jax.ShapeDtypeStruct((nv, 8 * F), jnp.float32),
            jax.ShapeDtypeStruct((nv, 8 * F), jnp.float32),
        ],
    )(degp8_v, xw_v, plo, phi)

    s1p_v = agg_call(t1_v.reshape(n, F), ei).reshape(NC, npv, 8 * F)

    b1p = jnp.zeros((F,), jnp.float32).at[:d_hid].set(b1)
    b1t = jnp.tile(b1p, 8).reshape(1, 8 * F)
    t2_v = pl.pallas_call(
        functools.partial(_tc2_body, nv),
        out_shape=jax.ShapeDtypeStruct((nv, 8 * F), jnp.float32),
    )(s1p_v, t1_v, dinv_v, b1t)

    s2p_v = agg_call(t2_v.reshape(n, F), ei).reshape(NC, npv, 8 * F)

    w2p = jnp.zeros((F, d_out), jnp.float32).at[:d_hid, :].set(W2)
    w2bd = _block_diag(w2p, 8)
    b2t = jnp.tile(b2, 8).reshape(1, 8 * d_out)
    sel = _block_diag(jnp.ones((d_out, 1), jnp.float32), 8)
    selt = _block_diag(jnp.ones((1, d_out), jnp.float32), 8)
    out8 = pl.pallas_call(
        functools.partial(_tc3_body, nv),
        out_shape=jax.ShapeDtypeStruct((nv, 8 * d_out), jnp.float32),
    )(s2p_v, t2_v, dinv_v, w2bd, b2t, sel, selt)
    return out8.reshape(n, d_out)


# speedup vs baseline: 1.1475x; 1.1475x over previous
import functools

import jax
import jax.numpy as jnp
from jax import lax
from jax.experimental import pallas as pl
from jax.experimental.pallas import tpu as pltpu
from jax.experimental.pallas import tpu_sc as plsc

NC = 2
NS = 16
NW = NC * NS
F = 16

_MESH = plsc.VectorSubcoreMesh(
    core_axis_name="c", subcore_axis_name="s", num_cores=NC, num_subcores=NS
)
_SC_PARAMS = pltpu.CompilerParams(use_tc_tiling_on_sc=False)


def _zero_rows(buf, nrows):
    zero = jnp.zeros((F,), jnp.float32)

    def body(j, _):
        buf[j] = zero
        return 0

    lax.fori_loop(0, nrows, body, 0, unroll=4)


def _make_agg_kernel(n_nodes, n_edges, chunk):
    epw = n_edges // NW
    nchunk = epw // chunk
    rpt = -(-n_nodes // (NS * 8)) * 8
    n_pad = rpt * NS

    def body(table, ei, out, sidx0, didx0, rows0, sidx1, didx1, rows1,
             tbuf, acc, gsem0, gsem1, ssem0, ssem1):
        c = lax.axis_index("c")
        s = lax.axis_index("s")
        wid = s * NC + c
        _zero_rows(tbuf, rpt)
        pltpu.sync_copy(tbuf, acc.at[pl.ds(s * rpt, rpt)])
        plsc.subcore_barrier()
        base = wid * epw
        bufs = [(sidx0, didx0, rows0, gsem0, ssem0),
                (sidx1, didx1, rows1, gsem1, ssem1)]

        def start_gather(k):
            sidx, didx, rows, gsem, _ = bufs[k % 2]
            off = base + k * chunk
            pltpu.sync_copy(ei.at[0].at[pl.ds(off, chunk)], sidx)
            pltpu.sync_copy(ei.at[1].at[pl.ds(off, chunk)], didx)
            return pltpu.async_copy(table.at[sidx], rows, gsem)

        gathers = {0: start_gather(0)}
        scatters = {}
        for k in range(nchunk):
            sidx, didx, rows, gsem, ssem = bufs[k % 2]
            if k + 1 < nchunk:
                if k - 1 >= 0:
                    scatters.pop(k - 1).wait()
                gathers[k + 1] = start_gather(k + 1)
            gathers.pop(k).wait()
            scatters[k] = pltpu.async_copy(rows, acc.at[didx], ssem, add=True)
        for d in scatters.values():
            d.wait()
        plsc.subcore_barrier()
        pltpu.sync_copy(acc.at[pl.ds(s * rpt, rpt)], tbuf)
        pltpu.sync_copy(tbuf, out.at[c].at[pl.ds(s * rpt, rpt)])

    return pl.kernel(
        body,
        out_type=jax.ShapeDtypeStruct((NC, n_pad, F), jnp.float32),
        mesh=_MESH,
        compiler_params=_SC_PARAMS,
        scratch_types=[
            pltpu.VMEM((chunk,), jnp.int32),
            pltpu.VMEM((chunk,), jnp.int32),
            pltpu.VMEM((chunk, F), jnp.float32),
            pltpu.VMEM((chunk,), jnp.int32),
            pltpu.VMEM((chunk,), jnp.int32),
            pltpu.VMEM((chunk, F), jnp.float32),
            pltpu.VMEM((rpt, F), jnp.float32),
            pltpu.VMEM_SHARED((n_pad, F), jnp.float32),
            pltpu.SemaphoreType.DMA,
            pltpu.SemaphoreType.DMA,
            pltpu.SemaphoreType.DMA,
            pltpu.SemaphoreType.DMA,
        ],
    )


DEGW = 16


def _make_deg_kernel(n_nodes, n_edges, chunk):
    epw = n_edges // NW
    nchunk = epw // chunk
    rpt = -(-n_nodes // (NS * 8)) * 8
    n_pad = rpt * NS

    def body(ei, out, didx0, didx1, rows, tbuf, acc, ssem0, ssem1):
        c = lax.axis_index("c")
        s = lax.axis_index("s")
        wid = s * NC + c
        _zero_rows(tbuf, rpt)
        pltpu.sync_copy(tbuf, acc.at[pl.ds(s * rpt, rpt)])
        one = jnp.ones((F,), jnp.float32)

        def fill(j, _):
            rows[j] = one
            return 0

        lax.fori_loop(0, chunk, fill, 0, unroll=4)
        plsc.subcore_barrier()
        base = wid * epw
        bufs = [(didx0, ssem0), (didx1, ssem1)]

        def load_idx(k):
            didx, _ = bufs[k % 2]
            pltpu.sync_copy(ei.at[1].at[pl.ds(base + k * chunk, chunk)], didx)

        load_idx(0)
        scatters = {}
        for k in range(nchunk):
            didx, ssem = bufs[k % 2]
            if k + 1 < nchunk:
                if k - 1 >= 0:
                    scatters.pop(k - 1).wait()
                load_idx(k + 1)
            scatters[k] = pltpu.async_copy(rows, acc.at[didx], ssem, add=True)
        for d in scatters.values():
            d.wait()
        plsc.subcore_barrier()
        pltpu.sync_copy(acc.at[pl.ds(s * rpt, rpt)], tbuf)
        pltpu.sync_copy(tbuf, out.at[c].at[pl.ds(s * rpt, rpt)])

    return pl.kernel(
        body,
        out_type=jax.ShapeDtypeStruct((NC, n_pad, DEGW), jnp.float32),
        mesh=_MESH,
        compiler_params=_SC_PARAMS,
        scratch_types=[
            pltpu.VMEM((chunk,), jnp.int32),
            pltpu.VMEM((chunk,), jnp.int32),
            pltpu.VMEM((chunk, DEGW), jnp.float32),
            pltpu.VMEM((rpt, DEGW), jnp.float32),
            pltpu.VMEM_SHARED((n_pad, DEGW), jnp.float32),
            pltpu.SemaphoreType.DMA,
            pltpu.SemaphoreType.DMA,
        ],
    )


def _tc1a_body(x8_ref, w1bd_ref, xw_ref):
    xw_ref[...] = jnp.dot(x8_ref[...], w1bd_ref[...],
                          preferred_element_type=jnp.float32)


def _tc1b_body(nv, degp_ref, xw_ref, t1_ref, dinv_ref):
    deg_v = degp_ref[0, :nv] + degp_ref[1, :nv] + 1.0
    dinv_v = lax.rsqrt(deg_v)
    dinv_ref[...] = dinv_v
    t1_ref[...] = xw_ref[...] * dinv_v


def _tc2_body(nv, aggp_ref, t1_ref, dinv_ref, b1_ref, t2_ref):
    dinv_v = dinv_ref[...]
    s_v = aggp_ref[0, :nv] + aggp_ref[1, :nv] + t1_ref[...]
    h_v = jnp.maximum(dinv_v * s_v + b1_ref[...], 0.0)
    t2_ref[...] = h_v * dinv_v


def _tc3_body(nv, aggp_ref, t2_ref, dinv_ref, w2bd_ref, b2_ref, sel_ref,
              selt_ref, out_ref):
    agg_v = dinv_ref[...] * (aggp_ref[0, :nv] + aggp_ref[1, :nv] + t2_ref[...])
    logits = (
        jnp.dot(agg_v, w2bd_ref[...], preferred_element_type=jnp.float32)
        + b2_ref[...]
    )
    m = jnp.max(logits, axis=1, keepdims=True)
    z = logits - m
    gsum = jnp.dot(jnp.exp(z), sel_ref[...], preferred_element_type=jnp.float32)
    lse = jnp.dot(jnp.log(gsum), selt_ref[...],
                  preferred_element_type=jnp.float32)
    z2 = z - lse
    gsum2 = jnp.dot(jnp.exp(z2), sel_ref[...],
                    preferred_element_type=jnp.float32)
    lse2 = jnp.dot(jnp.log(gsum2), selt_ref[...],
                   preferred_element_type=jnp.float32)
    out_ref[...] = z2 - lse2


def _block_diag(w, copies):
    d_in, d_out = w.shape
    out = jnp.zeros((copies * d_in, copies * d_out), w.dtype)
    for j in range(copies):
        out = out.at[j * d_in:(j + 1) * d_in, j * d_out:(j + 1) * d_out].set(w)
    return out


@jax.jit
def kernel(x, edge_index, W1, b1, W2, b2):
    n, d_in = x.shape
    d_hid = W1.shape[1]
    d_out = W2.shape[1]
    e = edge_index.shape[1]
    nv = n // 8

    ei = edge_index.astype(jnp.int32)

    chunk = 2000
    deg_call = _make_deg_kernel(n, e, chunk)
    agg_call = _make_agg_kernel(n, e, chunk)
    rpt = -(-n // (NS * 8)) * 8
    n_pad = rpt * NS
    npv = n_pad // 8
    nvh = n // 16

    degp_v = deg_call(ei).reshape(NC, n_pad * DEGW // 128, 128)

    w1p = jnp.zeros((d_in, F), W1.dtype).at[:, :d_hid].set(W1)
    w1bd = _block_diag(w1p, 8)
    x8 = x.reshape(nv, 8 * d_in)
    xw_v = pl.pallas_call(
        _tc1a_body,
        out_shape=jax.ShapeDtypeStruct((nv, 8 * F), jnp.float32),
    )(x8, w1bd)

    t1_v, dinv_v = pl.pallas_call(
        functools.partial(_tc1b_body, nv),
        out_shape=[
            jax.ShapeDtypeStruct((nv, 8 * F), jnp.float32),
            jax.ShapeDtypeStruct((nv, 8 * F), jnp.float32),
        ],
    )(degp_v, xw_v)

    s1p_v = agg_call(t1_v.reshape(n, F), ei).reshape(NC, npv, 8 * F)

    b1p = jnp.zeros((F,), jnp.float32).at[:d_hid].set(b1)
    b1t = jnp.tile(b1p, 8).reshape(1, 8 * F)
    t2_v = pl.pallas_call(
        functools.partial(_tc2_body, nv),
        out_shape=jax.ShapeDtypeStruct((nv, 8 * F), jnp.float32),
    )(s1p_v, t1_v, dinv_v, b1t)

    s2p_v = agg_call(t2_v.reshape(n, F), ei).reshape(NC, npv, 8 * F)

    w2p = jnp.zeros((F, d_out), jnp.float32).at[:d_hid, :].set(W2)
    w2bd = _block_diag(w2p, 8)
    b2t = jnp.tile(b2, 8).reshape(1, 8 * d_out)
    sel = _block_diag(jnp.ones((d_out, 1), jnp.float32), 8)
    selt = _block_diag(jnp.ones((1, d_out), jnp.float32), 8)
    out8 = pl.pallas_call(
        functools.partial(_tc3_body, nv),
        out_shape=jax.ShapeDtypeStruct((nv, 8 * d_out), jnp.float32),
    )(s2p_v, t2_v, dinv_v, w2bd, b2t, sel, selt)
    return out8.reshape(n, d_out)


# speedup vs baseline: 1.1688x; 1.0185x over previous
import functools

import jax
import jax.numpy as jnp
from jax import lax
from jax.experimental import pallas as pl
from jax.experimental.pallas import tpu as pltpu
from jax.experimental.pallas import tpu_sc as plsc

NC = 2
NS = 16
NW = NC * NS
F = 16

_MESH = plsc.VectorSubcoreMesh(
    core_axis_name="c", subcore_axis_name="s", num_cores=NC, num_subcores=NS
)
_SC_PARAMS = pltpu.CompilerParams(use_tc_tiling_on_sc=False)


def _zero_rows(buf, nrows):
    zero = jnp.zeros((F,), jnp.float32)

    def body(j, _):
        buf[j] = zero
        return 0

    lax.fori_loop(0, nrows, body, 0, unroll=4)


def _make_agg_kernel(n_nodes, n_edges, chunk):
    epw = n_edges // NW
    nchunk = epw // chunk
    rpt = -(-n_nodes // (NS * 8)) * 8
    n_pad = rpt * NS

    def body(table, ei, out, sidx0, didx0, rows0, sidx1, didx1, rows1,
             tbuf, acc, gsem0, gsem1, ssem0, ssem1):
        c = lax.axis_index("c")
        s = lax.axis_index("s")
        wid = s * NC + c
        _zero_rows(tbuf, rpt)
        pltpu.sync_copy(tbuf, acc.at[pl.ds(s * rpt, rpt)])
        plsc.subcore_barrier()
        base = wid * epw
        bufs = [(sidx0, didx0, rows0, gsem0, ssem0),
                (sidx1, didx1, rows1, gsem1, ssem1)]

        def start_gather(k):
            sidx, didx, rows, gsem, _ = bufs[k % 2]
            off = base + k * chunk
            pltpu.sync_copy(ei.at[0].at[pl.ds(off, chunk)], sidx)
            pltpu.sync_copy(ei.at[1].at[pl.ds(off, chunk)], didx)
            return pltpu.async_copy(table.at[sidx], rows, gsem)

        gathers = {0: start_gather(0)}
        scatters = {}
        for k in range(nchunk):
            sidx, didx, rows, gsem, ssem = bufs[k % 2]
            if k + 1 < nchunk:
                if k - 1 >= 0:
                    scatters.pop(k - 1).wait()
                gathers[k + 1] = start_gather(k + 1)
            gathers.pop(k).wait()
            scatters[k] = pltpu.async_copy(rows, acc.at[didx], ssem, add=True)
        for d in scatters.values():
            d.wait()
        plsc.subcore_barrier()
        pltpu.sync_copy(acc.at[pl.ds(s * rpt, rpt)], tbuf)
        pltpu.sync_copy(tbuf, out.at[c].at[pl.ds(s * rpt, rpt)])

    return pl.kernel(
        body,
        out_type=jax.ShapeDtypeStruct((NC, n_pad, F), jnp.float32),
        mesh=_MESH,
        compiler_params=_SC_PARAMS,
        scratch_types=[
            pltpu.VMEM((chunk,), jnp.int32),
            pltpu.VMEM((chunk,), jnp.int32),
            pltpu.VMEM((chunk, F), jnp.float32),
            pltpu.VMEM((chunk,), jnp.int32),
            pltpu.VMEM((chunk,), jnp.int32),
            pltpu.VMEM((chunk, F), jnp.float32),
            pltpu.VMEM((rpt, F), jnp.float32),
            pltpu.VMEM_SHARED((n_pad, F), jnp.float32),
            pltpu.SemaphoreType.DMA,
            pltpu.SemaphoreType.DMA,
            pltpu.SemaphoreType.DMA,
            pltpu.SemaphoreType.DMA,
        ],
    )


DEGW = 16


def _make_deg_kernel(n_nodes, n_edges, chunk):
    epw = n_edges // NW
    nchunk = epw // chunk
    rpt = -(-n_nodes // (NS * 8)) * 8
    n_pad = rpt * NS

    def body(ei, out, didx0, didx1, rows, tbuf, acc, ssem0, ssem1):
        c = lax.axis_index("c")
        s = lax.axis_index("s")
        wid = s * NC + c
        _zero_rows(tbuf, rpt)
        pltpu.sync_copy(tbuf, acc.at[pl.ds(s * rpt, rpt)])
        one = jnp.ones((F,), jnp.float32)

        def fill(j, _):
            rows[j] = one
            return 0

        lax.fori_loop(0, chunk, fill, 0, unroll=4)
        plsc.subcore_barrier()
        base = wid * epw
        bufs = [(didx0, ssem0), (didx1, ssem1)]

        def load_idx(k):
            didx, _ = bufs[k % 2]
            pltpu.sync_copy(ei.at[1].at[pl.ds(base + k * chunk, chunk)], didx)

        load_idx(0)
        scatters = {}
        for k in range(nchunk):
            didx, ssem = bufs[k % 2]
            if k + 1 < nchunk:
                if k - 1 >= 0:
                    scatters.pop(k - 1).wait()
                load_idx(k + 1)
            scatters[k] = pltpu.async_copy(rows, acc.at[didx], ssem, add=True)
        for d in scatters.values():
            d.wait()
        plsc.subcore_barrier()
        pltpu.sync_copy(acc.at[pl.ds(s * rpt, rpt)], tbuf)
        pltpu.sync_copy(tbuf, out.at[c].at[pl.ds(s * rpt, rpt)])

    return pl.kernel(
        body,
        out_type=jax.ShapeDtypeStruct((NC, n_pad, DEGW), jnp.float32),
        mesh=_MESH,
        compiler_params=_SC_PARAMS,
        scratch_types=[
            pltpu.VMEM((chunk,), jnp.int32),
            pltpu.VMEM((chunk,), jnp.int32),
            pltpu.VMEM((chunk, DEGW), jnp.float32),
            pltpu.VMEM((rpt, DEGW), jnp.float32),
            pltpu.VMEM_SHARED((n_pad, DEGW), jnp.float32),
            pltpu.SemaphoreType.DMA,
            pltpu.SemaphoreType.DMA,
        ],
    )


def _tc1_body(nv, x8_ref, w1bd_ref, degp_ref, t1_ref, dinv_ref):
    xw_v = jnp.dot(x8_ref[...], w1bd_ref[...],
                   preferred_element_type=jnp.float32)
    deg_v = degp_ref[0, :nv] + degp_ref[1, :nv] + 1.0
    dinv_v = lax.rsqrt(deg_v)
    dinv_ref[...] = dinv_v
    t1_ref[...] = xw_v * dinv_v


def _tc2_body(nv, aggp_ref, t1_ref, dinv_ref, b1_ref, t2_ref):
    dinv_v = dinv_ref[...]
    s_v = aggp_ref[0, :nv] + aggp_ref[1, :nv] + t1_ref[...]
    h_v = jnp.maximum(dinv_v * s_v + b1_ref[...], 0.0)
    t2_ref[...] = h_v * dinv_v


def _tc3_body(nv, aggp_ref, t2_ref, dinv_ref, w2bd_ref, b2_ref, sel_ref,
              selt_ref, out_ref):
    agg_v = dinv_ref[...] * (aggp_ref[0, :nv] + aggp_ref[1, :nv] + t2_ref[...])
    logits = (
        jnp.dot(agg_v, w2bd_ref[...], preferred_element_type=jnp.float32)
        + b2_ref[...]
    )
    m = jnp.max(logits, axis=1, keepdims=True)
    z = logits - m
    gsum = jnp.dot(jnp.exp(z), sel_ref[...], preferred_element_type=jnp.float32)
    lse = jnp.dot(jnp.log(gsum), selt_ref[...],
                  preferred_element_type=jnp.float32)
    z2 = z - lse
    gsum2 = jnp.dot(jnp.exp(z2), sel_ref[...],
                    preferred_element_type=jnp.float32)
    lse2 = jnp.dot(jnp.log(gsum2), selt_ref[...],
                   preferred_element_type=jnp.float32)
    out_ref[...] = z2 - lse2


def _block_diag(w, copies):
    d_in, d_out = w.shape
    out = jnp.zeros((copies * d_in, copies * d_out), w.dtype)
    for j in range(copies):
        out = out.at[j * d_in:(j + 1) * d_in, j * d_out:(j + 1) * d_out].set(w)
    return out


@jax.jit
def kernel(x, edge_index, W1, b1, W2, b2):
    n, d_in = x.shape
    d_hid = W1.shape[1]
    d_out = W2.shape[1]
    e = edge_index.shape[1]
    nv = n // 8

    ei = edge_index.astype(jnp.int32)

    chunk = 2000
    deg_call = _make_deg_kernel(n, e, chunk)
    agg_call = _make_agg_kernel(n, e, chunk)
    rpt = -(-n // (NS * 8)) * 8
    n_pad = rpt * NS
    npv = n_pad // 8
    nvh = n // 16

    degp_v = deg_call(ei).reshape(NC, n_pad * DEGW // 128, 128)

    w1p = jnp.zeros((d_in, F), W1.dtype).at[:, :d_hid].set(W1)
    w1bd = _block_diag(w1p, 8)
    x8 = x.reshape(nv, 8 * d_in)
    t1_v, dinv_v = pl.pallas_call(
        functools.partial(_tc1_body, nv),
        out_shape=[
            jax.ShapeDtypeStruct((nv, 8 * F), jnp.float32),
            jax.ShapeDtypeStruct((nv, 8 * F), jnp.float32),
        ],
    )(x8, w1bd, degp_v)

    s1p_v = agg_call(t1_v.reshape(n, F), ei).reshape(NC, npv, 8 * F)

    b1p = jnp.zeros((F,), jnp.float32).at[:d_hid].set(b1)
    b1t = jnp.tile(b1p, 8).reshape(1, 8 * F)
    t2_v = pl.pallas_call(
        functools.partial(_tc2_body, nv),
        out_shape=jax.ShapeDtypeStruct((nv, 8 * F), jnp.float32),
    )(s1p_v, t1_v, dinv_v, b1t)

    s2p_v = agg_call(t2_v.reshape(n, F), ei).reshape(NC, npv, 8 * F)

    w2p = jnp.zeros((F, d_out), jnp.float32).at[:d_hid, :].set(W2)
    w2bd = _block_diag(w2p, 8)
    b2t = jnp.tile(b2, 8).reshape(1, 8 * d_out)
    sel = _block_diag(jnp.ones((d_out, 1), jnp.float32), 8)
    selt = _block_diag(jnp.ones((1, d_out), jnp.float32), 8)
    out8 = pl.pallas_call(
        functools.partial(_tc3_body, nv),
        out_shape=jax.ShapeDtypeStruct((nv, 8 * d_out), jnp.float32),
    )(s2p_v, t2_v, dinv_v, w2bd, b2t, sel, selt)
    return out8.reshape(n, d_out)
